# baseline (speedup 1.0000x reference)
import jax
import jax.numpy as jnp
import numpy as np
from jax.experimental import pallas as pl
from jax.experimental.pallas import tpu as pltpu
from jax.sharding import Mesh, NamedSharding, PartitionSpec

P_WEIGHT = 1.0
U_WEIGHT = 0.5


def _shift_p_down(x):
    z = jnp.zeros_like(x[:, :1, :])
    return jnp.concatenate([z, x[:, :-1, :]], axis=1)


def _shift_p_up(x):
    z = jnp.zeros_like(x[:, :1, :])
    return jnp.concatenate([x[:, 1:, :], z], axis=1)


def _shift_c_right(x):
    z = jnp.zeros_like(x[:, :, :1])
    return jnp.concatenate([z, x[:, :, :-1]], axis=2)


def _shift_c_left(x):
    z = jnp.zeros_like(x[:, :, :1])
    return jnp.concatenate([x[:, :, 1:], z], axis=2)


def _hmax(x):
    return jnp.maximum(_shift_c_right(x), _shift_c_left(x))


def _lsmp_kernel(x_ref, out_ref):
    e = x_ref[:, :, :128]
    o = x_ref[:, :, 128:]
    shape = e.shape
    col = jax.lax.broadcasted_iota(jnp.int32, shape, 2)
    codd_b = (col & 1) == 1
    codd = codd_b.astype(e.dtype)
    ceven = 1.0 - codd
    codd_u = U_WEIGHT * codd
    ceven_u = U_WEIGHT * ceven

    w = jnp.where(codd_b, o, e)
    pv = jnp.where(codd_b,
                   jnp.maximum(_shift_p_down(o), o),
                   jnp.maximum(e, _shift_p_up(e)))
    vm = jnp.maximum(_hmax(w), pv)
    e = e - codd * vm
    o = o - ceven * vm

    w = jnp.where(codd_b, e, o)
    pv = jnp.where(codd_b,
                   jnp.maximum(e, _shift_p_up(e)),
                   jnp.maximum(_shift_p_down(o), o))
    vm = jnp.maximum(_hmax(w), pv)
    e = e + ceven_u * vm
    o = o + codd_u * vm

    hm_e = _hmax(e)
    dm_o = jnp.maximum(hm_e, _shift_p_up(hm_e))
    o = o - codd * dm_o

    hm_o = _hmax(o)
    dm_e = jnp.maximum(hm_o, _shift_p_down(hm_o))
    out_e = e + U_WEIGHT * dm_e

    idx = 2 * jax.lax.broadcasted_iota(jnp.int32, (shape[0], shape[1], shape[2] // 2), 2)
    out_ref[...] = jnp.take_along_axis(out_e, idx, axis=2)


def _lsmp(x4, block):
    n = x4.shape[0]
    return pl.pallas_call(
        _lsmp_kernel,
        grid=(n // block,),
        in_specs=[pl.BlockSpec((block, 64, 256), lambda i: (i, 0, 0))],
        out_specs=pl.BlockSpec((block, 64, 64), lambda i: (i, 0, 0)),
        out_shape=jax.ShapeDtypeStruct((n, 64, 64), x4.dtype),
        compiler_params=pltpu.CompilerParams(
            dimension_semantics=("parallel",),
        ),
    )(x4)


def _shard_fn(t):
    b, c, h, w = t.shape
    t4 = t.reshape(b * c, h // 2, 2 * w)
    out = _lsmp(t4, block=16)
    return out.reshape(b, c, h // 2, w // 2)


def kernel(x):
    devs = jax.devices()
    if len(devs) >= 2:
        mesh = Mesh(np.array(devs[:2]), ("d",))
        xs = jax.device_put(x, NamedSharding(mesh, PartitionSpec("d")))
        return jax.shard_map(
            _shard_fn, mesh=mesh,
            in_specs=PartitionSpec("d"), out_specs=PartitionSpec("d"),
            check_vma=False,
        )(xs)
    return _shard_fn(x)


# speedup vs baseline: 1.1594x; 1.1594x over previous
import jax
import jax.numpy as jnp
from jax.experimental import pallas as pl
from jax.experimental.pallas import tpu as pltpu

P_WEIGHT = 1.0
U_WEIGHT = 0.5


def _shift_p_down(x):
    z = jnp.zeros_like(x[:, :1, :])
    return jnp.concatenate([z, x[:, :-1, :]], axis=1)


def _shift_p_up(x):
    z = jnp.zeros_like(x[:, :1, :])
    return jnp.concatenate([x[:, 1:, :], z], axis=1)


def _shift_c_right(x):
    z = jnp.zeros_like(x[:, :, :1])
    return jnp.concatenate([z, x[:, :, :-1]], axis=2)


def _shift_c_left(x):
    z = jnp.zeros_like(x[:, :, :1])
    return jnp.concatenate([x[:, :, 1:], z], axis=2)


def _hmax(x):
    return jnp.maximum(_shift_c_right(x), _shift_c_left(x))


def _lsmp_kernel(x_ref, out_ref):
    e = x_ref[:, :, :128]
    o = x_ref[:, :, 128:]
    shape = e.shape
    col = jax.lax.broadcasted_iota(jnp.int32, shape, 2)
    codd_b = (col & 1) == 1
    codd = codd_b.astype(e.dtype)
    ceven = 1.0 - codd
    codd_u = U_WEIGHT * codd
    ceven_u = U_WEIGHT * ceven

    w = jnp.where(codd_b, o, e)
    pv = jnp.where(codd_b,
                   jnp.maximum(_shift_p_down(o), o),
                   jnp.maximum(e, _shift_p_up(e)))
    vm = jnp.maximum(_hmax(w), pv)
    e = e - codd * vm
    o = o - ceven * vm

    w = jnp.where(codd_b, e, o)
    pv = jnp.where(codd_b,
                   jnp.maximum(e, _shift_p_up(e)),
                   jnp.maximum(_shift_p_down(o), o))
    vm = jnp.maximum(_hmax(w), pv)
    e = e + ceven_u * vm
    o = o + codd_u * vm

    hm_e = _hmax(e)
    dm_o = jnp.maximum(hm_e, _shift_p_up(hm_e))
    o = o - codd * dm_o

    hm_o = _hmax(o)
    dm_e = jnp.maximum(hm_o, _shift_p_down(hm_o))
    out_e = e + U_WEIGHT * dm_e

    idx = 2 * jax.lax.broadcasted_iota(jnp.int32, (shape[0], shape[1], shape[2] // 2), 2)
    out_ref[...] = jnp.take_along_axis(out_e, idx, axis=2)


def _lsmp(x4, block):
    n = x4.shape[0]
    return pl.pallas_call(
        _lsmp_kernel,
        grid=(n // block,),
        in_specs=[pl.BlockSpec((block, 64, 256), lambda i: (i, 0, 0))],
        out_specs=pl.BlockSpec((block, 64, 64), lambda i: (i, 0, 0)),
        out_shape=jax.ShapeDtypeStruct((n, 64, 64), x4.dtype),
        compiler_params=pltpu.CompilerParams(
            dimension_semantics=("parallel",),
        ),
    )(x4)


def kernel(x):
    b, c, h, w = x.shape
    n_chunks = 4 if b % 4 == 0 else 1
    outs = []
    for ch in jnp.split(x, n_chunks, axis=0):
        cb = ch.shape[0]
        t4 = ch.reshape(cb * c, h // 2, 2 * w)
        outs.append(_lsmp(t4, block=16).reshape(cb, c, h // 2, w // 2))
    return jnp.concatenate(outs, axis=0) if n_chunks > 1 else outs[0]


# speedup vs baseline: 1.9021x; 1.6407x over previous
import jax
import jax.numpy as jnp
from jax.experimental import pallas as pl
from jax.experimental.pallas import tpu as pltpu

P_WEIGHT = 1.0
U_WEIGHT = 0.5


def _shift_p_down(x):
    z = jnp.zeros_like(x[:, :1, :])
    return jnp.concatenate([z, x[:, :-1, :]], axis=1)


def _shift_p_up(x):
    z = jnp.zeros_like(x[:, :1, :])
    return jnp.concatenate([x[:, 1:, :], z], axis=1)


def _shift_c_right(x):
    z = jnp.zeros_like(x[:, :, :1])
    return jnp.concatenate([z, x[:, :, :-1]], axis=2)


def _shift_c_left(x):
    z = jnp.zeros_like(x[:, :, :1])
    return jnp.concatenate([x[:, :, 1:], z], axis=2)


def _hmax(x):
    return jnp.maximum(_shift_c_right(x), _shift_c_left(x))


def _lsmp_kernel(x_ref, out_ref):
    x = x_ref[...]
    b = x.shape[0]
    z = x.reshape(b, 16, 8, 128)
    s = jax.lax.broadcasted_iota(jnp.int32, z.shape, 2)
    sidx = 2 * (s & 3) + (s >> 2)
    g = jnp.take_along_axis(z, sidx, axis=2)
    e = g[:, :, 0:4, :].reshape(b, 64, 128)
    o = g[:, :, 4:8, :].reshape(b, 64, 128)
    shape = e.shape
    col = jax.lax.broadcasted_iota(jnp.int32, shape, 2)
    codd_b = (col & 1) == 1
    codd = codd_b.astype(e.dtype)
    ceven = 1.0 - codd
    codd_u = U_WEIGHT * codd
    ceven_u = U_WEIGHT * ceven

    w = jnp.where(codd_b, o, e)
    pv = jnp.where(codd_b,
                   jnp.maximum(_shift_p_down(o), o),
                   jnp.maximum(e, _shift_p_up(e)))
    vm = jnp.maximum(_hmax(w), pv)
    e = e - codd * vm
    o = o - ceven * vm

    w = jnp.where(codd_b, e, o)
    pv = jnp.where(codd_b,
                   jnp.maximum(e, _shift_p_up(e)),
                   jnp.maximum(_shift_p_down(o), o))
    vm = jnp.maximum(_hmax(w), pv)
    e = e + ceven_u * vm
    o = o + codd_u * vm

    hm_e = _hmax(e)
    dm_o = jnp.maximum(hm_e, _shift_p_up(hm_e))
    o = o - codd * dm_o

    hm_o = _hmax(o)
    dm_e = jnp.maximum(hm_o, _shift_p_down(hm_o))
    out_e = e + U_WEIGHT * dm_e

    idx = 2 * jax.lax.broadcasted_iota(jnp.int32, (shape[0], shape[1], shape[2] // 2), 2)
    out_ref[...] = jnp.take_along_axis(out_e, idx, axis=2)


def _lsmp(x3, block):
    n = x3.shape[0]
    return pl.pallas_call(
        _lsmp_kernel,
        grid=(n // block,),
        in_specs=[pl.BlockSpec((block, 128, 128), lambda i: (i, 0, 0))],
        out_specs=pl.BlockSpec((block, 64, 64), lambda i: (i, 0, 0)),
        out_shape=jax.ShapeDtypeStruct((n, 64, 64), x3.dtype),
        compiler_params=pltpu.CompilerParams(
            dimension_semantics=("parallel",),
        ),
    )(x3)


def kernel(x):
    b, c, h, w = x.shape
    x3 = x.reshape(b * c, h, w)
    out = _lsmp(x3, block=16)
    return out.reshape(b, c, h // 2, w // 2)
